# baseline (speedup 1.0000x reference)
import functools

import jax
import jax.numpy as jnp
from jax import lax
from jax.experimental import pallas as pl
from jax.experimental.pallas import tpu as pltpu
from jax.experimental.pallas import tpu_sc as plsc

N = 50000
E = 800000
H = 64
L = 8
GATE_VALS = (3, 2, 5, 1, 4)
NG = 5
NP = 3
RW = 128

QSIZE = 12504
SHQ = 12544
DUMP = 12520
ZPT = SHQ // 16
WB_PT = 776
WB_TAIL = QSIZE - 16 * WB_PT
MSG_ROWS = 3 * QSIZE + QSIZE
EPT = E // 16
ECH = 2000
CCAP = 2176
BR = 128

NB = 400
NBLK = N // NB
NBG = 2000
NBLKG = N // NBG

_mesh = plsc.VectorSubcoreMesh(core_axis_name="c", subcore_axis_name="s")
_sc_params = pltpu.CompilerParams(needs_layout_passes=False)


SUB = E // 32
NKEY = 32
BTW = 48
EPAD = E + 1024


@functools.partial(
    pl.kernel, mesh=_mesh,
    compiler_params=_sc_params,
    out_type=(jax.ShapeDtypeStruct((E,), jnp.int32),
              jax.ShapeDtypeStruct((E,), jnp.int32),
              jax.ShapeDtypeStruct((E,), jnp.int32),
              jax.ShapeDtypeStruct((32 * 512,), jnp.int32),
              jax.ShapeDtypeStruct((32 * BTW,), jnp.int32)),
    scratch_types=[
        pltpu.VMEM((N,), jnp.int32),
        pltpu.VMEM((N,), jnp.int32),
        pltpu.VMEM((ECH,), jnp.int32),
        pltpu.VMEM((ECH,), jnp.int32),
        pltpu.VMEM((ECH,), jnp.int32),
        pltpu.VMEM((ECH,), jnp.int32),
        pltpu.VMEM((ECH,), jnp.int32),
        pltpu.VMEM((512,), jnp.int32),
        pltpu.VMEM((BTW,), jnp.int32),
        pltpu.SemaphoreType.DMA,
    ],
)
def _prep1_kernel(src_hbm, dst_hbm, lvl_hbm, gate_hbm,
                  key_hbm, ridx_hbm, dloc_hbm, offtab_hbm, btab_hbm,
                  lvlbuf, gatebuf, es, ed, okey, orr, odl, hist, obt, sem):
    c = lax.axis_index("c")
    s = lax.axis_index("s")
    tid = s * 2 + c
    pltpu.sync_copy(lvl_hbm, lvlbuf)
    pltpu.sync_copy(gate_hbm, gatebuf)
    base = tid * SUB
    iota = lax.iota(jnp.int32, 16)
    ones = jnp.ones((16,), jnp.int32)

    def zh(i, _):
        hist[pl.ds(i * 16, 16)] = jnp.zeros((16,), jnp.int32)
        return 0

    lax.fori_loop(0, 32, zh, 0)

    def do_vreg(d16, s16, msk):
        d16c = jnp.minimum(jnp.maximum(d16, 0), N - 1)
        lvv = plsc.load_gather(lvlbuf, [d16c])
        gv = plsc.load_gather(gatebuf, [d16c])
        p = jnp.where((gv == 2) | (gv == 3), 0,
            jnp.where((gv == 1) | (gv == 5), 1,
            jnp.where(gv == 4, 2, 0)))
        valid = (gv >= 1) & (gv <= 5)
        lv_eff = jnp.where(valid, lvv, 0)
        qt = ((d16c >= QSIZE).astype(jnp.int32)
              + (d16c >= 2 * QSIZE).astype(jnp.int32)
              + (d16c >= 3 * QSIZE).astype(jnp.int32))
        k = lv_eff * 4 + qt
        plsc.addupdate_scatter(hist, [k * 16 + iota], ones, mask=msk)
        return k, p * N + s16, d16 - qt * QSIZE

    tmask = jnp.full((16,), True)

    def emit(j, msk):
        d16 = ed[pl.ds(j * 16, 16)]
        s16 = es[pl.ds(j * 16, 16)]
        k, r, dl = do_vreg(d16, s16, msk)
        okey[pl.ds(j * 16, 16)] = k
        orr[pl.ds(j * 16, 16)] = r
        odl[pl.ds(j * 16, 16)] = dl

    def full_chunk(ci, _):
        off = ci * ECH
        pltpu.sync_copy(src_hbm.at[pl.ds(base + off, ECH)], es)
        pltpu.sync_copy(dst_hbm.at[pl.ds(base + off, ECH)], ed)

        def vb(j, _):
            emit(j, tmask)
            return 0

        lax.fori_loop(0, ECH // 16, vb, 0)
        pltpu.sync_copy(okey, key_hbm.at[pl.ds(base + off, ECH)])
        pltpu.sync_copy(orr, ridx_hbm.at[pl.ds(base + off, ECH)])
        pltpu.sync_copy(odl, dloc_hbm.at[pl.ds(base + off, ECH)])
        return 0

    lax.fori_loop(0, 12, full_chunk, 0)
    toff = 12 * ECH
    pltpu.sync_copy(src_hbm.at[pl.ds(base + toff, 1000)], es.at[pl.ds(0, 1000)])
    pltpu.sync_copy(dst_hbm.at[pl.ds(base + toff, 1000)], ed.at[pl.ds(0, 1000)])

    def vbt(j, _):
        emit(j, tmask)
        return 0

    lax.fori_loop(0, 62, vbt, 0)
    emit(62, iota < 8)
    pltpu.sync_copy(okey.at[pl.ds(0, 1000)], key_hbm.at[pl.ds(base + toff, 1000)])
    pltpu.sync_copy(orr.at[pl.ds(0, 1000)], ridx_hbm.at[pl.ds(base + toff, 1000)])
    pltpu.sync_copy(odl.at[pl.ds(0, 1000)], dloc_hbm.at[pl.ds(base + toff, 1000)])

    def off_k(k, carry):
        v = hist[pl.ds(k * 16, 16)]
        cs = plsc.cumsum(v)
        excl = cs - v + carry
        hist[pl.ds(k * 16, 16)] = excl
        plsc.store_scatter(obt, [jnp.full((16,), k, jnp.int32)],
                           jnp.zeros((16,), jnp.int32) + carry,
                           mask=iota == 0)
        return carry + jnp.sum(v)

    carry = lax.fori_loop(0, 32, off_k, jnp.int32(0))
    plsc.store_scatter(obt, [jnp.full((16,), 32, jnp.int32)],
                       jnp.zeros((16,), jnp.int32) + carry, mask=iota == 0)
    pltpu.sync_copy(hist, offtab_hbm.at[pl.ds(tid * 512, 512)])
    pltpu.sync_copy(obt, btab_hbm.at[pl.ds(tid * BTW, BTW)])


@functools.partial(
    pl.kernel, mesh=_mesh,
    compiler_params=_sc_params,
    out_type=(jax.ShapeDtypeStruct((EPAD,), jnp.int32),
              jax.ShapeDtypeStruct((EPAD,), jnp.int32)),
    scratch_types=[
        pltpu.VMEM((ECH,), jnp.int32),
        pltpu.VMEM((ECH,), jnp.int32),
        pltpu.VMEM((ECH,), jnp.int32),
        pltpu.VMEM((512,), jnp.int32),
        pltpu.VMEM((SUB,), jnp.int32),
        pltpu.VMEM((SUB,), jnp.int32),
        pltpu.VMEM((1024,), jnp.int32),
        pltpu.SemaphoreType.DMA,
    ],
)
def _prep2_kernel(key_hbm, ridx_hbm, dloc_hbm, offtab_hbm,
                  rs_hbm, dls_hbm,
                  ckey, cri, cdl, curoffs, rvs, dvs, zbuf, sem):
    c = lax.axis_index("c")
    s = lax.axis_index("s")
    tid = s * 2 + c
    base = tid * SUB
    iota = lax.iota(jnp.int32, 16)
    ones = jnp.ones((16,), jnp.int32)
    pltpu.sync_copy(offtab_hbm.at[pl.ds(tid * 512, 512)], curoffs)

    def place(j, msk):
        k16 = ckey[pl.ds(j * 16, 16)]
        r16 = cri[pl.ds(j * 16, 16)]
        dl16 = cdl[pl.ds(j * 16, 16)]
        slot = k16 * 16 + iota
        slot = jnp.minimum(jnp.maximum(slot, 0), 511)
        pos = plsc.load_gather(curoffs, [slot], mask=msk)
        plsc.addupdate_scatter(curoffs, [slot], ones, mask=msk)
        pos = jnp.minimum(jnp.maximum(pos, 0), SUB - 1)
        plsc.store_scatter(rvs, [pos], r16, mask=msk)
        plsc.store_scatter(dvs, [pos], dl16, mask=msk)

    tmask = jnp.full((16,), True)

    def full_chunk(ci, _):
        off = ci * ECH
        pltpu.sync_copy(key_hbm.at[pl.ds(base + off, ECH)], ckey)
        pltpu.sync_copy(ridx_hbm.at[pl.ds(base + off, ECH)], cri)
        pltpu.sync_copy(dloc_hbm.at[pl.ds(base + off, ECH)], cdl)

        def vb(j, _):
            place(j, tmask)
            return 0

        lax.fori_loop(0, ECH // 16, vb, 0)
        return 0

    lax.fori_loop(0, 12, full_chunk, 0)
    toff = 12 * ECH
    pltpu.sync_copy(key_hbm.at[pl.ds(base + toff, 1000)], ckey.at[pl.ds(0, 1000)])
    pltpu.sync_copy(ridx_hbm.at[pl.ds(base + toff, 1000)], cri.at[pl.ds(0, 1000)])
    pltpu.sync_copy(dloc_hbm.at[pl.ds(base + toff, 1000)], cdl.at[pl.ds(0, 1000)])

    def vbt(j, _):
        place(j, tmask)
        return 0

    lax.fori_loop(0, 62, vbt, 0)
    place(62, iota < 8)

    pltpu.sync_copy(rvs, rs_hbm.at[pl.ds(base, SUB)])
    pltpu.sync_copy(dvs, dls_hbm.at[pl.ds(base, SUB)])

    @pl.when(tid == 31)
    def _():
        def zz(i, _):
            zbuf[pl.ds(i * 16, 16)] = jnp.zeros((16,), jnp.int32)
            return 0

        lax.fori_loop(0, 64, zz, 0)
        pltpu.sync_copy(zbuf, rs_hbm.at[pl.ds(E, 1024)])
        pltpu.sync_copy(zbuf, dls_hbm.at[pl.ds(E, 1024)])


@functools.partial(
    pl.kernel, mesh=_mesh,
    compiler_params=_sc_params,
    out_type=jax.ShapeDtypeStruct((MSG_ROWS, RW), jnp.float32),
    scratch_types=[
        pltpu.VMEM_SHARED((SHQ, RW), jnp.float32),
        pltpu.VMEM((32 * BTW,), jnp.int32),
        pltpu.VMEM((1024,), jnp.int32),
        pltpu.VMEM((1024,), jnp.int32),
        pltpu.VMEM((BR, RW), jnp.float32),
        pltpu.VMEM((16,), jnp.int32),
        pltpu.SemaphoreType.DMA,
    ],
)
def _edge_kernel(rs_hbm, dls_hbm, btab_hbm, lvl_hbm, r2_hbm, msg_hbm,
                 msgsh, btabv, er, ed, rows32, lvl16, sem):
    c = lax.axis_index("c")
    s = lax.axis_index("s")
    iota = lax.iota(jnp.int32, 16)
    pltpu.sync_copy(btab_hbm, btabv)
    pltpu.sync_copy(lvl_hbm, lvl16)
    lv = lvl16[...]
    level_s = lv[0]

    def zr(i, _):
        for col in range(RW // 16):
            rows32[i, pl.ds(col * 16, 16)] = jnp.zeros((16,), jnp.float32)
        return 0

    for q in range(2):
        qidx = c * 2 + q
        key = level_s * 4 + qidx

        lax.fori_loop(0, BR, zr, 0)
        zbase = s * ZPT

        def zb(b, _):
            pltpu.sync_copy(rows32, msgsh.at[pl.ds(zbase + b * BR, BR)])
            return 0

        lax.fori_loop(0, 6, zb, 0)
        pltpu.sync_copy(rows32.at[pl.ds(0, 16)],
                        msgsh.at[pl.ds(zbase + 768, 16)])
        plsc.subcore_barrier()

        for rsub in range(2):
            r = s * 2 + rsub
            v = btabv[pl.ds(r * BTW + key, 16)]
            g0 = r * SUB + v[0]
            g1 = r * SUB + v[1]
            a0 = (g0 // 8) * 8
            nch = (g1 - a0 + 1023) // 1024

            def ch(ci, _):
                cb = a0 + ci * 1024
                pltpu.sync_copy(rs_hbm.at[pl.ds(cb, 1024)], er)
                pltpu.sync_copy(dls_hbm.at[pl.ds(cb, 1024)], ed)
                nbat = (jnp.minimum(g1 - cb, 1024) + BR - 1) // BR

                def bt(b, _):
                    boff = b * BR
                    pltpu.async_copy(r2_hbm.at[er.at[pl.ds(boff, BR)]],
                                     rows32, sem).wait()
                    for t in range(BR // 16):
                        d16 = ed[pl.ds(boff + t * 16, 16)]
                        posv = cb + boff + t * 16 + iota
                        okm = (posv >= g0) & (posv < g1)
                        dadj = jnp.where(okm, d16, DUMP)
                        pltpu.sync_copy(rows32.at[pl.ds(t * 16, 16)],
                                        msgsh.at[dadj], add=True)
                    return 0

                lax.fori_loop(0, nbat, bt, 0)
                return 0

            lax.fori_loop(0, nch, ch, 0)

        plsc.subcore_barrier()

        qlo = qidx * QSIZE
        wb = s * WB_PT
        pltpu.sync_copy(msgsh.at[pl.ds(wb, WB_PT)],
                        msg_hbm.at[pl.ds(qlo + wb, WB_PT)])

        @pl.when(s == 0)
        def _():
            pltpu.sync_copy(msgsh.at[pl.ds(16 * WB_PT, WB_TAIL)],
                            msg_hbm.at[pl.ds(qlo + 16 * WB_PT, WB_TAIL)])

        plsc.subcore_barrier()


def _hs_body(xmg_ref, wst_ref, whs_ref, bhs_ref, out_ref):
    t6 = jnp.dot(wst_ref[...], whs_ref[...],
                 preferred_element_type=jnp.float32) + bhs_ref[...]
    xm = xmg_ref[...]
    acc = jnp.zeros((NB, H), jnp.float32)
    for v in range(6):
        acc = jnp.where(xm == v, t6[v:v + 1, :], acc)
    out_ref[...] = acc


def _z_body(hs_ref, hf_ref, was_ref, waf_ref, b_ref, out_ref):
    acc = (jnp.dot(hs_ref[...], was_ref[...], preferred_element_type=jnp.float32)
           + jnp.dot(hf_ref[...], waf_ref[...], preferred_element_type=jnp.float32)
           + b_ref[...])
    out_ref[...] = jnp.maximum(acc, 0.0)[None, :, :]


def _gru_body(lvl_s, msg_ref, hf_ref, gate_ref, flvl_ref, wi_ref, wh_ref,
              bi_ref, bh_ref, out_ref):
    level = lvl_s[0]
    h = hf_ref[...]
    g = gate_ref[...]
    mr = msg_ref[...]
    m = jnp.zeros((NBG, H), jnp.float32)
    for gidx, gval in enumerate(GATE_VALS):
        half = (gidx % 2) * H
        m = jnp.where(g == gval, mr[:, half:half + H], m)
    gi_all = jnp.dot(m, wi_ref[...], preferred_element_type=jnp.float32)
    gh_all = jnp.dot(h, wh_ref[...], preferred_element_type=jnp.float32)
    gi = jnp.zeros((NBG, 3 * H), jnp.float32)
    gh = jnp.zeros((NBG, 3 * H), jnp.float32)
    for gidx, gval in enumerate(GATE_VALS):
        sel = g == gval
        gi = jnp.where(sel, gi_all[:, gidx * 192:(gidx + 1) * 192]
                       + bi_ref[gidx:gidx + 1, :], gi)
        gh = jnp.where(sel, gh_all[:, gidx * 192:(gidx + 1) * 192]
                       + bh_ref[gidx:gidx + 1, :], gh)
    ir, iz, inn = gi[:, :H], gi[:, H:2 * H], gi[:, 2 * H:]
    hr, hz, hn = gh[:, :H], gh[:, H:2 * H], gh[:, 2 * H:]
    r = jax.nn.sigmoid(ir + hr)
    z = jax.nn.sigmoid(iz + hz)
    n = jnp.tanh(inn + r * hn)
    hnew = (1.0 - z) * n + z * h
    active = (flvl_ref[...] == level) & (g >= 1) & (g <= 5)
    out_ref[...] = jnp.where(active, hnew, h)


def _hs_call(xmg2, wst_p, w_hs, b_hs2):
    return pl.pallas_call(
        _hs_body,
        grid=(NBLK,),
        in_specs=[
            pl.BlockSpec((NB, 1), lambda i: (i, 0)),
            pl.BlockSpec((8, 2 * H), lambda i: (0, 0)),
            pl.BlockSpec((2 * H, H), lambda i: (0, 0)),
            pl.BlockSpec((1, H), lambda i: (0, 0)),
        ],
        out_specs=pl.BlockSpec((NB, H), lambda i: (i, 0)),
        out_shape=jax.ShapeDtypeStruct((N, H), jnp.float32),
    )(xmg2, wst_p, w_hs, b_hs2)


def _z_call(hs, hf, was, waf, bcat):
    return pl.pallas_call(
        _z_body,
        grid=(NP, NBLK),
        in_specs=[
            pl.BlockSpec((NB, H), lambda p, i: (i, 0)),
            pl.BlockSpec((NB, H), lambda p, i: (i, 0)),
            pl.BlockSpec((H, RW), lambda p, i: (0, p)),
            pl.BlockSpec((H, RW), lambda p, i: (0, p)),
            pl.BlockSpec((1, RW), lambda p, i: (0, p)),
        ],
        out_specs=pl.BlockSpec((1, NB, RW), lambda p, i: (p, i, 0)),
        out_shape=jax.ShapeDtypeStruct((NP, N, RW), jnp.float32),
    )(hs, hf, was, waf, bcat)


def _gru_call(lvl_arr, msg, hf, gate, flvl2, wi_r, wh_r, bi_p, bh_p):
    return pl.pallas_call(
        _gru_body,
        grid=(NBLKG,),
        in_specs=[
            pl.BlockSpec(memory_space=pltpu.SMEM),
            pl.BlockSpec((NBG, RW), lambda i: (i, 0)),
            pl.BlockSpec((NBG, H), lambda i: (i, 0)),
            pl.BlockSpec((NBG, 1), lambda i: (i, 0)),
            pl.BlockSpec((NBG, 1), lambda i: (i, 0)),
            pl.BlockSpec((H, 15 * H), lambda i: (0, 0)),
            pl.BlockSpec((H, 15 * H), lambda i: (0, 0)),
            pl.BlockSpec((8, 3 * H), lambda i: (0, 0)),
            pl.BlockSpec((8, 3 * H), lambda i: (0, 0)),
        ],
        out_specs=pl.BlockSpec((NBG, H), lambda i: (i, 0)),
        out_shape=jax.ShapeDtypeStruct((N, H), jnp.float32),
    )(lvl_arr, msg, hf, gate, flvl2, wi_r, wh_r, bi_p, bh_p)


def kernel(x, edge_index, gate, xmg_x, forward_level, forward_index,
           Ws, Wt, W_hs, b_hs, W_aggr, b_aggr, Wi, Wh, bi, bh):
    src = edge_index[0]
    dst = edge_index[1]
    gate_flat = gate[:, 0]
    xmg2 = xmg_x[:, 1:2]
    flvl2 = forward_level.reshape(N, 1)

    wst_p = jnp.zeros((8, 2 * H), jnp.float32).at[:6].set(
        jnp.concatenate([Ws, Wt], axis=1))
    b_hs2 = b_hs.reshape(1, H)
    wcat = jnp.zeros((2 * H, NP * RW), jnp.float32).at[:, :NG * H].set(
        jnp.transpose(W_aggr, (1, 0, 2)).reshape(2 * H, NG * H))
    was, waf = wcat[:H], wcat[H:]
    bcat = jnp.zeros((1, NP * RW), jnp.float32).at[:, :NG * H].set(
        b_aggr.reshape(1, NG * H))
    wi_r = jnp.transpose(Wi, (1, 0, 2)).reshape(H, NG * 3 * H)
    wh_r = jnp.transpose(Wh, (1, 0, 2)).reshape(H, NG * 3 * H)
    bi_p = jnp.zeros((8, 3 * H), jnp.float32).at[:NG].set(bi)
    bh_p = jnp.zeros((8, 3 * H), jnp.float32).at[:NG].set(bh)

    hs = _hs_call(xmg2, wst_p, W_hs, b_hs2)
    key_e, ridx_e, dloc_e, offtab, btab = _prep1_kernel(
        src, dst, forward_level, gate_flat)
    ridx_s, dloc_s = _prep2_kernel(key_e, ridx_e, dloc_e, offtab)

    hf = jnp.zeros((N, H), jnp.float32)
    for level in range(1, L):
        r = _z_call(hs, hf, was, waf, bcat)
        r2 = r.reshape(NP * N, RW)
        lvl16 = jnp.full((16,), level, jnp.int32)
        msg = _edge_kernel(ridx_s, dloc_s, btab, lvl16, r2)
        lvl_arr = jnp.full((1,), level, jnp.int32)
        hf = _gru_call(lvl_arr, msg, hf, gate, flvl2, wi_r, wh_r,
                       bi_p, bh_p)

    return (hs, hf)


# speedup vs baseline: 1.1523x; 1.1523x over previous
import functools

import jax
import jax.numpy as jnp
from jax import lax
from jax.experimental import pallas as pl
from jax.experimental.pallas import tpu as pltpu
from jax.experimental.pallas import tpu_sc as plsc

N = 50000
E = 800000
H = 64
L = 8
GATE_VALS = (3, 2, 5, 1, 4)
NG = 5
NP = 3
RW = 128

QSIZE = 12504
SHQ = 12544
DUMP = 12520
ZPT = SHQ // 16
WB_PT = 776
WB_TAIL = QSIZE - 16 * WB_PT
MSG_ROWS = 3 * QSIZE + QSIZE
EPT = E // 16
ECH = 2000
CCAP = 2176
BR = 128

NB = 400
NBLK = N // NB
NBG = 2000
NBLKG = N // NBG

_mesh = plsc.VectorSubcoreMesh(core_axis_name="c", subcore_axis_name="s")
_sc_params = pltpu.CompilerParams(needs_layout_passes=False)


SUB = E // 32
NKEY = 32
BTW = 48
EPAD = E + 1024


@functools.partial(
    pl.kernel, mesh=_mesh,
    compiler_params=_sc_params,
    out_type=(jax.ShapeDtypeStruct((E,), jnp.int32),
              jax.ShapeDtypeStruct((E,), jnp.int32),
              jax.ShapeDtypeStruct((E,), jnp.int32),
              jax.ShapeDtypeStruct((32 * 512,), jnp.int32),
              jax.ShapeDtypeStruct((32 * BTW,), jnp.int32)),
    scratch_types=[
        pltpu.VMEM((N,), jnp.int32),
        pltpu.VMEM((N,), jnp.int32),
        pltpu.VMEM((ECH,), jnp.int32),
        pltpu.VMEM((ECH,), jnp.int32),
        pltpu.VMEM((ECH,), jnp.int32),
        pltpu.VMEM((ECH,), jnp.int32),
        pltpu.VMEM((ECH,), jnp.int32),
        pltpu.VMEM((512,), jnp.int32),
        pltpu.VMEM((BTW,), jnp.int32),
        pltpu.SemaphoreType.DMA,
    ],
)
def _prep1_kernel(src_hbm, dst_hbm, lvl_hbm, gate_hbm,
                  key_hbm, ridx_hbm, dloc_hbm, offtab_hbm, btab_hbm,
                  lvlbuf, gatebuf, es, ed, okey, orr, odl, hist, obt, sem):
    c = lax.axis_index("c")
    s = lax.axis_index("s")
    tid = s * 2 + c
    pltpu.sync_copy(lvl_hbm, lvlbuf)
    pltpu.sync_copy(gate_hbm, gatebuf)
    base = tid * SUB
    iota = lax.iota(jnp.int32, 16)
    ones = jnp.ones((16,), jnp.int32)

    def zh(i, _):
        hist[pl.ds(i * 16, 16)] = jnp.zeros((16,), jnp.int32)
        return 0

    lax.fori_loop(0, 32, zh, 0)

    def do_vreg(d16, s16, msk):
        d16c = jnp.minimum(jnp.maximum(d16, 0), N - 1)
        lvv = plsc.load_gather(lvlbuf, [d16c])
        gv = plsc.load_gather(gatebuf, [d16c])
        p = jnp.where((gv == 2) | (gv == 3), 0,
            jnp.where((gv == 1) | (gv == 5), 1,
            jnp.where(gv == 4, 2, 0)))
        valid = (gv >= 1) & (gv <= 5)
        lv_eff = jnp.where(valid, lvv, 0)
        qt = ((d16c >= QSIZE).astype(jnp.int32)
              + (d16c >= 2 * QSIZE).astype(jnp.int32)
              + (d16c >= 3 * QSIZE).astype(jnp.int32))
        k = lv_eff * 4 + qt
        plsc.addupdate_scatter(hist, [k * 16 + iota], ones, mask=msk)
        return k, s16 * 3 + p, d16 - qt * QSIZE

    tmask = jnp.full((16,), True)

    def emit(j, msk):
        d16 = ed[pl.ds(j * 16, 16)]
        s16 = es[pl.ds(j * 16, 16)]
        k, r, dl = do_vreg(d16, s16, msk)
        okey[pl.ds(j * 16, 16)] = k
        orr[pl.ds(j * 16, 16)] = r
        odl[pl.ds(j * 16, 16)] = dl

    def full_chunk(ci, _):
        off = ci * ECH
        pltpu.sync_copy(src_hbm.at[pl.ds(base + off, ECH)], es)
        pltpu.sync_copy(dst_hbm.at[pl.ds(base + off, ECH)], ed)

        def vb(j, _):
            emit(j, tmask)
            return 0

        lax.fori_loop(0, ECH // 16, vb, 0)
        pltpu.sync_copy(okey, key_hbm.at[pl.ds(base + off, ECH)])
        pltpu.sync_copy(orr, ridx_hbm.at[pl.ds(base + off, ECH)])
        pltpu.sync_copy(odl, dloc_hbm.at[pl.ds(base + off, ECH)])
        return 0

    lax.fori_loop(0, 12, full_chunk, 0)
    toff = 12 * ECH
    pltpu.sync_copy(src_hbm.at[pl.ds(base + toff, 1000)], es.at[pl.ds(0, 1000)])
    pltpu.sync_copy(dst_hbm.at[pl.ds(base + toff, 1000)], ed.at[pl.ds(0, 1000)])

    def vbt(j, _):
        emit(j, tmask)
        return 0

    lax.fori_loop(0, 62, vbt, 0)
    emit(62, iota < 8)
    pltpu.sync_copy(okey.at[pl.ds(0, 1000)], key_hbm.at[pl.ds(base + toff, 1000)])
    pltpu.sync_copy(orr.at[pl.ds(0, 1000)], ridx_hbm.at[pl.ds(base + toff, 1000)])
    pltpu.sync_copy(odl.at[pl.ds(0, 1000)], dloc_hbm.at[pl.ds(base + toff, 1000)])

    def off_k(k, carry):
        v = hist[pl.ds(k * 16, 16)]
        cs = plsc.cumsum(v)
        excl = cs - v + carry
        hist[pl.ds(k * 16, 16)] = excl
        plsc.store_scatter(obt, [jnp.full((16,), k, jnp.int32)],
                           jnp.zeros((16,), jnp.int32) + carry,
                           mask=iota == 0)
        return carry + jnp.sum(v)

    carry = lax.fori_loop(0, 32, off_k, jnp.int32(0))
    plsc.store_scatter(obt, [jnp.full((16,), 32, jnp.int32)],
                       jnp.zeros((16,), jnp.int32) + carry, mask=iota == 0)
    pltpu.sync_copy(hist, offtab_hbm.at[pl.ds(tid * 512, 512)])
    pltpu.sync_copy(obt, btab_hbm.at[pl.ds(tid * BTW, BTW)])


@functools.partial(
    pl.kernel, mesh=_mesh,
    compiler_params=_sc_params,
    out_type=(jax.ShapeDtypeStruct((EPAD,), jnp.int32),
              jax.ShapeDtypeStruct((EPAD,), jnp.int32)),
    scratch_types=[
        pltpu.VMEM((ECH,), jnp.int32),
        pltpu.VMEM((ECH,), jnp.int32),
        pltpu.VMEM((ECH,), jnp.int32),
        pltpu.VMEM((512,), jnp.int32),
        pltpu.VMEM((SUB,), jnp.int32),
        pltpu.VMEM((SUB,), jnp.int32),
        pltpu.VMEM((1024,), jnp.int32),
        pltpu.SemaphoreType.DMA,
    ],
)
def _prep2_kernel(key_hbm, ridx_hbm, dloc_hbm, offtab_hbm,
                  rs_hbm, dls_hbm,
                  ckey, cri, cdl, curoffs, rvs, dvs, zbuf, sem):
    c = lax.axis_index("c")
    s = lax.axis_index("s")
    tid = s * 2 + c
    base = tid * SUB
    iota = lax.iota(jnp.int32, 16)
    ones = jnp.ones((16,), jnp.int32)
    pltpu.sync_copy(offtab_hbm.at[pl.ds(tid * 512, 512)], curoffs)

    def place(j, msk):
        k16 = ckey[pl.ds(j * 16, 16)]
        r16 = cri[pl.ds(j * 16, 16)]
        dl16 = cdl[pl.ds(j * 16, 16)]
        slot = k16 * 16 + iota
        slot = jnp.minimum(jnp.maximum(slot, 0), 511)
        pos = plsc.load_gather(curoffs, [slot], mask=msk)
        plsc.addupdate_scatter(curoffs, [slot], ones, mask=msk)
        pos = jnp.minimum(jnp.maximum(pos, 0), SUB - 1)
        plsc.store_scatter(rvs, [pos], r16, mask=msk)
        plsc.store_scatter(dvs, [pos], dl16, mask=msk)

    tmask = jnp.full((16,), True)

    def full_chunk(ci, _):
        off = ci * ECH
        pltpu.sync_copy(key_hbm.at[pl.ds(base + off, ECH)], ckey)
        pltpu.sync_copy(ridx_hbm.at[pl.ds(base + off, ECH)], cri)
        pltpu.sync_copy(dloc_hbm.at[pl.ds(base + off, ECH)], cdl)

        def vb(j, _):
            place(j, tmask)
            return 0

        lax.fori_loop(0, ECH // 16, vb, 0)
        return 0

    lax.fori_loop(0, 12, full_chunk, 0)
    toff = 12 * ECH
    pltpu.sync_copy(key_hbm.at[pl.ds(base + toff, 1000)], ckey.at[pl.ds(0, 1000)])
    pltpu.sync_copy(ridx_hbm.at[pl.ds(base + toff, 1000)], cri.at[pl.ds(0, 1000)])
    pltpu.sync_copy(dloc_hbm.at[pl.ds(base + toff, 1000)], cdl.at[pl.ds(0, 1000)])

    def vbt(j, _):
        place(j, tmask)
        return 0

    lax.fori_loop(0, 62, vbt, 0)
    place(62, iota < 8)

    pltpu.sync_copy(rvs, rs_hbm.at[pl.ds(base, SUB)])
    pltpu.sync_copy(dvs, dls_hbm.at[pl.ds(base, SUB)])

    @pl.when(tid == 31)
    def _():
        def zz(i, _):
            zbuf[pl.ds(i * 16, 16)] = jnp.zeros((16,), jnp.int32)
            return 0

        lax.fori_loop(0, 64, zz, 0)
        pltpu.sync_copy(zbuf, rs_hbm.at[pl.ds(E, 1024)])
        pltpu.sync_copy(zbuf, dls_hbm.at[pl.ds(E, 1024)])


@functools.partial(
    pl.kernel, mesh=_mesh,
    compiler_params=_sc_params,
    out_type=jax.ShapeDtypeStruct((MSG_ROWS, RW), jnp.float32),
    scratch_types=[
        pltpu.VMEM_SHARED((SHQ, RW), jnp.float32),
        pltpu.VMEM((32 * BTW,), jnp.int32),
        pltpu.VMEM((1024,), jnp.int32),
        pltpu.VMEM((1024,), jnp.int32),
        pltpu.VMEM((BR, RW), jnp.float32),
        pltpu.VMEM((16,), jnp.int32),
        pltpu.SemaphoreType.DMA,
    ],
)
def _edge_kernel(rs_hbm, dls_hbm, btab_hbm, lvl_hbm, r2_hbm, msg_hbm,
                 msgsh, btabv, er, ed, rows32, lvl16, sem):
    c = lax.axis_index("c")
    s = lax.axis_index("s")
    iota = lax.iota(jnp.int32, 16)
    pltpu.sync_copy(btab_hbm, btabv)
    pltpu.sync_copy(lvl_hbm, lvl16)
    lv = lvl16[...]
    level_s = lv[0]

    def zr(i, _):
        for col in range(RW // 16):
            rows32[i, pl.ds(col * 16, 16)] = jnp.zeros((16,), jnp.float32)
        return 0

    for q in range(2):
        qidx = c * 2 + q
        key = level_s * 4 + qidx

        lax.fori_loop(0, BR, zr, 0)
        zbase = s * ZPT

        def zb(b, _):
            pltpu.sync_copy(rows32, msgsh.at[pl.ds(zbase + b * BR, BR)])
            return 0

        lax.fori_loop(0, 6, zb, 0)
        pltpu.sync_copy(rows32.at[pl.ds(0, 16)],
                        msgsh.at[pl.ds(zbase + 768, 16)])
        plsc.subcore_barrier()

        for rsub in range(2):
            r = s * 2 + rsub
            v = btabv[pl.ds(r * BTW + key, 16)]
            g0 = r * SUB + v[0]
            g1 = r * SUB + v[1]
            a0 = (g0 // 8) * 8
            nch = (g1 - a0 + 1023) // 1024

            def ch(ci, _):
                cb = a0 + ci * 1024
                pltpu.sync_copy(rs_hbm.at[pl.ds(cb, 1024)], er)
                pltpu.sync_copy(dls_hbm.at[pl.ds(cb, 1024)], ed)
                nbat = (jnp.minimum(g1 - cb, 1024) + BR - 1) // BR

                def bt(b, _):
                    boff = b * BR
                    pltpu.async_copy(r2_hbm.at[er.at[pl.ds(boff, BR)]],
                                     rows32, sem).wait()
                    for t in range(BR // 16):
                        d16 = ed[pl.ds(boff + t * 16, 16)]
                        posv = cb + boff + t * 16 + iota
                        okm = (posv >= g0) & (posv < g1)
                        dadj = jnp.where(okm, d16, DUMP)
                        pltpu.sync_copy(rows32.at[pl.ds(t * 16, 16)],
                                        msgsh.at[dadj], add=True)
                    return 0

                lax.fori_loop(0, nbat, bt, 0)
                return 0

            lax.fori_loop(0, nch, ch, 0)

        plsc.subcore_barrier()

        qlo = qidx * QSIZE
        wb = s * WB_PT
        pltpu.sync_copy(msgsh.at[pl.ds(wb, WB_PT)],
                        msg_hbm.at[pl.ds(qlo + wb, WB_PT)])

        @pl.when(s == 0)
        def _():
            pltpu.sync_copy(msgsh.at[pl.ds(16 * WB_PT, WB_TAIL)],
                            msg_hbm.at[pl.ds(qlo + 16 * WB_PT, WB_TAIL)])

        plsc.subcore_barrier()


def _hs_body(xmg_ref, wst_ref, whs_ref, bhs_ref, out_ref):
    t6 = jnp.dot(wst_ref[...], whs_ref[...],
                 preferred_element_type=jnp.float32) + bhs_ref[...]
    xm = xmg_ref[...]
    acc = jnp.zeros((NB, H), jnp.float32)
    for v in range(6):
        acc = jnp.where(xm == v, t6[v:v + 1, :], acc)
    out_ref[...] = acc


def _z_body(hs_ref, hf_ref, was_ref, waf_ref, b_ref, out_ref):
    acc = (jnp.dot(hs_ref[...], was_ref[...], preferred_element_type=jnp.float32)
           + jnp.dot(hf_ref[...], waf_ref[...], preferred_element_type=jnp.float32)
           + b_ref[...])
    out_ref[...] = jnp.maximum(acc, 0.0)


def _gru_body(lvl_s, msg_ref, hf_ref, gate_ref, flvl_ref, wbig_ref,
              bi_ref, bh_ref, out_ref):
    level = lvl_s[0]
    h = hf_ref[...]
    g = gate_ref[...]
    mr = msg_ref[...]
    m = jnp.zeros((NBG, H), jnp.float32)
    for gidx, gval in enumerate(GATE_VALS):
        half = (gidx % 2) * H
        m = jnp.where(g == gval, mr[:, half:half + H], m)
    x = jnp.concatenate([m, h], axis=-1)
    y = jnp.dot(x, wbig_ref[...], preferred_element_type=jnp.float32)
    gi_all = y[:, :NG * 3 * H]
    gh_all = y[:, NG * 3 * H:]
    gi = jnp.zeros((NBG, 3 * H), jnp.float32)
    gh = jnp.zeros((NBG, 3 * H), jnp.float32)
    for gidx, gval in enumerate(GATE_VALS):
        sel = g == gval
        gi = jnp.where(sel, gi_all[:, gidx * 192:(gidx + 1) * 192]
                       + bi_ref[gidx:gidx + 1, :], gi)
        gh = jnp.where(sel, gh_all[:, gidx * 192:(gidx + 1) * 192]
                       + bh_ref[gidx:gidx + 1, :], gh)
    ir, iz, inn = gi[:, :H], gi[:, H:2 * H], gi[:, 2 * H:]
    hr, hz, hn = gh[:, :H], gh[:, H:2 * H], gh[:, 2 * H:]
    r = jax.nn.sigmoid(ir + hr)
    z = jax.nn.sigmoid(iz + hz)
    n = jnp.tanh(inn + r * hn)
    hnew = (1.0 - z) * n + z * h
    active = (flvl_ref[...] == level) & (g >= 1) & (g <= 5)
    out_ref[...] = jnp.where(active, hnew, h)


def _hs_call(xmg2, wst_p, w_hs, b_hs2):
    return pl.pallas_call(
        _hs_body,
        grid=(NBLK,),
        in_specs=[
            pl.BlockSpec((NB, 1), lambda i: (i, 0)),
            pl.BlockSpec((8, 2 * H), lambda i: (0, 0)),
            pl.BlockSpec((2 * H, H), lambda i: (0, 0)),
            pl.BlockSpec((1, H), lambda i: (0, 0)),
        ],
        out_specs=pl.BlockSpec((NB, H), lambda i: (i, 0)),
        out_shape=jax.ShapeDtypeStruct((N, H), jnp.float32),
    )(xmg2, wst_p, w_hs, b_hs2)


def _z_call(hs, hf, was, waf, bcat):
    return pl.pallas_call(
        _z_body,
        grid=(NBLK,),
        in_specs=[
            pl.BlockSpec((NB, H), lambda i: (i, 0)),
            pl.BlockSpec((NB, H), lambda i: (i, 0)),
            pl.BlockSpec((H, NP * RW), lambda i: (0, 0)),
            pl.BlockSpec((H, NP * RW), lambda i: (0, 0)),
            pl.BlockSpec((1, NP * RW), lambda i: (0, 0)),
        ],
        out_specs=pl.BlockSpec((NB, NP * RW), lambda i: (i, 0)),
        out_shape=jax.ShapeDtypeStruct((N, NP * RW), jnp.float32),
    )(hs, hf, was, waf, bcat)


def _gru_call(lvl_arr, msg, hf, gate, flvl2, wbig, bi_p, bh_p):
    return pl.pallas_call(
        _gru_body,
        grid=(NBLKG,),
        in_specs=[
            pl.BlockSpec(memory_space=pltpu.SMEM),
            pl.BlockSpec((NBG, RW), lambda i: (i, 0)),
            pl.BlockSpec((NBG, H), lambda i: (i, 0)),
            pl.BlockSpec((NBG, 1), lambda i: (i, 0)),
            pl.BlockSpec((NBG, 1), lambda i: (i, 0)),
            pl.BlockSpec((2 * H, 30 * H), lambda i: (0, 0)),
            pl.BlockSpec((8, 3 * H), lambda i: (0, 0)),
            pl.BlockSpec((8, 3 * H), lambda i: (0, 0)),
        ],
        out_specs=pl.BlockSpec((NBG, H), lambda i: (i, 0)),
        out_shape=jax.ShapeDtypeStruct((N, H), jnp.float32),
    )(lvl_arr, msg, hf, gate, flvl2, wbig, bi_p, bh_p)


def kernel(x, edge_index, gate, xmg_x, forward_level, forward_index,
           Ws, Wt, W_hs, b_hs, W_aggr, b_aggr, Wi, Wh, bi, bh):
    src = edge_index[0]
    dst = edge_index[1]
    gate_flat = gate[:, 0]
    xmg2 = xmg_x[:, 1:2]
    flvl2 = forward_level.reshape(N, 1)

    wst_p = jnp.zeros((8, 2 * H), jnp.float32).at[:6].set(
        jnp.concatenate([Ws, Wt], axis=1))
    b_hs2 = b_hs.reshape(1, H)
    wcat = jnp.zeros((2 * H, NP * RW), jnp.float32).at[:, :NG * H].set(
        jnp.transpose(W_aggr, (1, 0, 2)).reshape(2 * H, NG * H))
    was, waf = wcat[:H], wcat[H:]
    bcat = jnp.zeros((1, NP * RW), jnp.float32).at[:, :NG * H].set(
        b_aggr.reshape(1, NG * H))
    wi_r = jnp.transpose(Wi, (1, 0, 2)).reshape(H, NG * 3 * H)
    wh_r = jnp.transpose(Wh, (1, 0, 2)).reshape(H, NG * 3 * H)
    wbig = jnp.zeros((2 * H, 2 * NG * 3 * H), jnp.float32)
    wbig = wbig.at[:H, :NG * 3 * H].set(wi_r).at[H:, NG * 3 * H:].set(wh_r)
    bi_p = jnp.zeros((8, 3 * H), jnp.float32).at[:NG].set(bi)
    bh_p = jnp.zeros((8, 3 * H), jnp.float32).at[:NG].set(bh)

    hs = _hs_call(xmg2, wst_p, W_hs, b_hs2)
    key_e, ridx_e, dloc_e, offtab, btab = _prep1_kernel(
        src, dst, forward_level, gate_flat)
    ridx_s, dloc_s = _prep2_kernel(key_e, ridx_e, dloc_e, offtab)

    hf = jnp.zeros((N, H), jnp.float32)
    for level in range(1, L):
        r = _z_call(hs, hf, was, waf, bcat)
        r2 = r.reshape(NP * N, RW)
        lvl16 = jnp.full((16,), level, jnp.int32)
        msg = _edge_kernel(ridx_s, dloc_s, btab, lvl16, r2)
        lvl_arr = jnp.full((1,), level, jnp.int32)
        hf = _gru_call(lvl_arr, msg, hf, gate, flvl2, wbig, bi_p, bh_p)

    return (hs, hf)
